# Initial kernel scaffold; baseline (speedup 1.0000x reference)
import jax
import jax.numpy as jnp
from jax.experimental import pallas as pl


def kernel(input, table):
    raise NotImplementedError("write your pallas kernel here")


# speedup vs baseline: 1.0954x; 1.0954x over previous
import functools

import jax
import jax.numpy as jnp
from jax import lax
from jax.experimental import pallas as pl
from jax.experimental.pallas import tpu as pltpu
from jax.experimental.pallas import tpu_sc as plsc

EMBED_DIM = 32
NUM_CORES = 2
NUM_SUBCORES = 16
NUM_WORKERS = NUM_CORES * NUM_SUBCORES


def _build(n_rows: int, chunk: int):
    per_w = n_rows // NUM_WORKERS
    n_chunks = per_w // chunk
    assert per_w % chunk == 0

    mesh = plsc.VectorSubcoreMesh(core_axis_name="c", subcore_axis_name="s")

    @functools.partial(
        pl.kernel,
        mesh=mesh,
        out_type=jax.ShapeDtypeStruct((n_rows, EMBED_DIM), jnp.float32),
        scratch_types=[
            pltpu.VMEM((chunk,), jnp.int32),
            pltpu.VMEM((chunk, EMBED_DIM), jnp.float32),
            pltpu.SemaphoreType.DMA,
        ],
        compiler_params=pltpu.CompilerParams(use_tc_tiling_on_sc=False),
    )
    def emb(idx_hbm, table_hbm, out_hbm, idx_v, rows_v, sem):
        wid = lax.axis_index("s") * NUM_CORES + lax.axis_index("c")
        base = wid * per_w

        def body(i, carry):
            off = base + i * chunk
            pltpu.sync_copy(idx_hbm.at[pl.ds(off, chunk)], idx_v)
            pltpu.async_copy(table_hbm.at[idx_v], rows_v, sem).wait()
            pltpu.sync_copy(rows_v, out_hbm.at[pl.ds(off, chunk)])
            return carry

        lax.fori_loop(0, n_chunks, body, 0)

    return emb


def kernel(input, table):
    b, s = input.shape
    n_rows = b * s
    flat_idx = input.reshape(n_rows).astype(jnp.int32)
    emb = _build(n_rows, chunk=1024)
    out = emb(flat_idx, table)
    return out.reshape(b, s, EMBED_DIM)


# speedup vs baseline: 1.1098x; 1.0132x over previous
import functools

import jax
import jax.numpy as jnp
from jax import lax
from jax.experimental import pallas as pl
from jax.experimental.pallas import tpu as pltpu
from jax.experimental.pallas import tpu_sc as plsc

EMBED_DIM = 32
NUM_CORES = 2
NUM_SUBCORES = 16
NUM_WORKERS = NUM_CORES * NUM_SUBCORES
NBUF = 2


def _build(n_rows: int, chunk: int):
    per_w = n_rows // NUM_WORKERS
    n_chunks = per_w // chunk
    assert per_w % chunk == 0 and n_chunks % NBUF == 0 and n_chunks >= 2 * NBUF

    mesh = plsc.VectorSubcoreMesh(core_axis_name="c", subcore_axis_name="s")

    @functools.partial(
        pl.kernel,
        mesh=mesh,
        out_type=jax.ShapeDtypeStruct((n_rows, EMBED_DIM), jnp.float32),
        scratch_types=[
            pltpu.VMEM((NBUF, chunk), jnp.int32),
            pltpu.VMEM((NBUF, chunk, EMBED_DIM), jnp.float32),
            [pltpu.SemaphoreType.DMA] * NBUF,
            [pltpu.SemaphoreType.DMA] * NBUF,
            [pltpu.SemaphoreType.DMA] * NBUF,
        ],
        compiler_params=pltpu.CompilerParams(use_tc_tiling_on_sc=False),
    )
    def emb(idx_hbm, table_hbm, out_hbm, idx_v, rows_v, sem_idx, sem_g, sem_out):
        wid = lax.axis_index("s") * NUM_CORES + lax.axis_index("c")
        base = wid * per_w

        def start_idx(k, b):
            pltpu.async_copy(
                idx_hbm.at[pl.ds(base + k * chunk, chunk)], idx_v.at[b], sem_idx[b]
            )

        def gather(b):
            pltpu.async_copy(table_hbm.at[idx_v.at[b]], rows_v.at[b], sem_g[b])

        def wait_gather(b):
            pltpu.make_async_copy(table_hbm.at[idx_v.at[b]], rows_v.at[b], sem_g[b]).wait()

        def start_out(k, b):
            pltpu.async_copy(
                rows_v.at[b], out_hbm.at[pl.ds(base + k * chunk, chunk)], sem_out[b]
            )

        def wait_out(k, b):
            pltpu.make_async_copy(
                rows_v.at[b], out_hbm.at[pl.ds(base + k * chunk, chunk)], sem_out[b]
            ).wait()

        def wait_idx(k, b):
            pltpu.make_async_copy(
                idx_hbm.at[pl.ds(base + k * chunk, chunk)], idx_v.at[b], sem_idx[b]
            ).wait()

        for b in range(NBUF):
            start_idx(b, b)
        for k in range(NBUF):
            b = k
            wait_idx(k, b)
            gather(b)
            wait_gather(b)
            start_out(k, b)
            start_idx(k + NBUF, b)

        @pl.loop(NBUF, n_chunks - NBUF, step=NBUF)
        def _stage(g):
            for b in range(NBUF):
                k = g + b
                wait_idx(k, b)
                wait_out(k - NBUF, b)
                gather(b)
                wait_gather(b)
                start_out(k, b)
                start_idx(k + NBUF, b)

        for off in range(NBUF):
            b = off
            k_sym = n_chunks - NBUF + off
            wait_idx(k_sym, b)
            wait_out(k_sym - NBUF, b)
            gather(b)
            wait_gather(b)
            start_out(k_sym, b)
        for b in range(NBUF):
            wait_out(n_chunks - NBUF + b, b)

    return emb


def kernel(input, table):
    b, s = input.shape
    n_rows = b * s
    flat_idx = input.reshape(n_rows).astype(jnp.int32)
    emb = _build(n_rows, chunk=1600)
    out = emb(flat_idx, table)
    return out.reshape(b, s, EMBED_DIM)


# speedup vs baseline: 1.7949x; 1.6172x over previous
import functools

import jax
import jax.numpy as jnp
from jax import lax
from jax.experimental import pallas as pl
from jax.experimental.pallas import tpu as pltpu
from jax.experimental.pallas import tpu_sc as plsc

EMBED_DIM = 32
NUM_CORES = 2
NUM_SUBCORES = 16
NUM_WORKERS = NUM_CORES * NUM_SUBCORES
NBUF = 2


def _build(n_batch: int, seq: int, rows_blk: int):
    per_w = n_batch // NUM_WORKERS
    chunk = rows_blk * seq
    n_chunks = per_w // rows_blk
    assert per_w % rows_blk == 0 and n_chunks % NBUF == 0 and n_chunks >= 2 * NBUF

    mesh = plsc.VectorSubcoreMesh(core_axis_name="c", subcore_axis_name="s")

    @functools.partial(
        pl.kernel,
        mesh=mesh,
        out_type=jax.ShapeDtypeStruct((n_batch, seq, EMBED_DIM), jnp.float32),
        scratch_types=[
            pltpu.VMEM((NBUF, chunk), jnp.int32),
            pltpu.VMEM((NBUF, chunk, EMBED_DIM), jnp.float32),
            [pltpu.SemaphoreType.DMA] * NBUF,
            [pltpu.SemaphoreType.DMA] * NBUF,
            [pltpu.SemaphoreType.DMA] * NBUF,
        ],
        compiler_params=pltpu.CompilerParams(use_tc_tiling_on_sc=False),
    )
    def emb(idx_hbm, table_hbm, out_hbm, idx_v, rows_v, sem_idx, sem_g, sem_out):
        wid = lax.axis_index("s") * NUM_CORES + lax.axis_index("c")
        base = wid * per_w

        def idx_copy(k, b):
            return pltpu.make_async_copy(
                idx_hbm.at[pl.ds((base + k * rows_blk) * seq, chunk)],
                idx_v.at[b],
                sem_idx[b],
            )

        def gather_copy(b):
            return pltpu.make_async_copy(
                table_hbm.at[idx_v.at[b]], rows_v.at[b], sem_g[b]
            )

        def _row_copy(k, b, r):
            return pltpu.make_async_copy(
                rows_v.at[b].at[pl.ds(r * seq, seq)],
                out_hbm.at[base + k * rows_blk + r],
                sem_out[b],
            )

        class out_copy:
            def __init__(self, k, b):
                self.k, self.b = k, b

            def start(self):
                for r in range(rows_blk):
                    _row_copy(self.k, self.b, r).start()

            def wait(self):
                for r in range(rows_blk):
                    _row_copy(self.k, self.b, r).wait()

        for b in range(NBUF):
            idx_copy(b, b).start()
        for k in range(NBUF):
            b = k
            idx_copy(k, b).wait()
            gather_copy(b).start()
            gather_copy(b).wait()
            out_copy(k, b).start()
            idx_copy(k + NBUF, b).start()

        @pl.loop(NBUF, n_chunks - NBUF, step=NBUF)
        def _stage(g):
            for b in range(NBUF):
                k = g + b
                idx_copy(k, b).wait()
                out_copy(k - NBUF, b).wait()
                gather_copy(b).start()
                gather_copy(b).wait()
                out_copy(k, b).start()
                idx_copy(k + NBUF, b).start()

        for off in range(NBUF):
            b = off
            k = n_chunks - NBUF + off
            idx_copy(k, b).wait()
            out_copy(k - NBUF, b).wait()
            gather_copy(b).start()
            gather_copy(b).wait()
            out_copy(k, b).start()
        for b in range(NBUF):
            out_copy(n_chunks - NBUF + b, b).wait()

    return emb


def kernel(input, table):
    b, s = input.shape
    emb = _build(b, s, rows_blk=32)
    out = emb(input.reshape(b * s).astype(jnp.int32), table)
    return out
